# baseline (speedup 1.0000x reference)
import functools

import jax
import jax.numpy as jnp
from jax import lax
from jax.experimental import pallas as pl
from jax.experimental.pallas import tpu as pltpu
from jax.experimental.pallas import tpu_sc as plsc

NC = 2
NS = 16
NW = NC * NS
L = 16

B = 16384
D = 16
V = 1000000
BPW = B // NW
IDXC = 128
KCH = BPW // IDXC

NT = V // 128
ROWS = (NT + 1) * D
FLEN = ROWS * 128
SEGL = FLEN - (D - 1) * 128
TPS = (NT + NS - 1) // NS


def _detile_body(table_r, patch_r, out_r, sem, s):
    def tile(j, carry):
        t = jnp.minimum(s + NS * j, NT - 1)
        for dg in range(2):
            pltpu.async_copy(
                table_r.at[pl.ds(dg * 8, 8), pl.ds(t * 128, 128)],
                out_r.at[pl.ds(t * D + dg * 8, 8), :], sem)
        return carry

    lax.fori_loop(0, TPS, tile, 0)

    @pl.when(s == 0)
    def _patch():
        pltpu.async_copy(patch_r, out_r.at[pl.ds(NT * D, D), :], sem)
        pltpu.make_async_copy(out_r.at[pl.ds(0, D), :],
                              out_r.at[pl.ds(0, D), :], sem).wait()

    nrows = TPS * 2 * 8
    pltpu.make_async_copy(out_r.at[pl.ds(0, nrows), :],
                          out_r.at[pl.ds(0, nrows), :], sem).wait()


@functools.partial(
    pl.kernel,
    out_type=(jax.ShapeDtypeStruct((ROWS, 128), jnp.float32),
              jax.ShapeDtypeStruct((ROWS, 128), jnp.float32)),
    mesh=plsc.VectorSubcoreMesh(core_axis_name="c", subcore_axis_name="s"),
    compiler_params=pltpu.CompilerParams(needs_layout_passes=False),
    scratch_types=[
        pltpu.SemaphoreType.DMA,
    ],
)
def _sc_detile(ut_r, it_r, upatch_r, ipatch_r, ubi_r, ibi_r, sem):
    c = lax.axis_index("c")
    s = lax.axis_index("s")

    @pl.when(c == 0)
    def _users():
        _detile_body(ut_r, upatch_r, ubi_r, sem, s)

    @pl.when(c == 1)
    def _items():
        _detile_body(it_r, ipatch_r, ibi_r, sem, s)


def _gather_body(users_r, items_r, ut_r, it_r, out_r,
                 uidx, iidx, ucols, icols, outv, sem):
    w = lax.axis_index("s") * NC + lax.axis_index("c")
    base = w * BPW

    pltpu.sync_copy(users_r.at[pl.ds(base, BPW)], uidx)
    pltpu.sync_copy(items_r.at[pl.ds(base, BPW)], iidx)

    handles = []
    for d in range(D):
        useg = ut_r.at[pl.ds(d * 128, SEGL)]
        iseg = it_r.at[pl.ds(d * 128, SEGL)]
        for k in range(KCH):
            sl = pl.ds(k * IDXC, IDXC)
            handles.append(pltpu.async_copy(
                useg.at[uidx.at[sl]], ucols.at[d, sl], sem))
            handles.append(pltpu.async_copy(
                iseg.at[iidx.at[sl]], icols.at[d, sl], sem))
    for h in handles:
        h.wait()

    def block(j, carry):
        rb = pl.ds(j * L, L)
        acc = ucols[0, rb] * icols[0, rb]
        for d in range(1, D):
            acc = acc + ucols[d, rb] * icols[d, rb]
        outv[rb] = acc
        return carry

    lax.fori_loop(0, BPW // L, block, 0)

    pltpu.sync_copy(outv, out_r.at[pl.ds(base, BPW)])


@functools.partial(
    pl.kernel,
    out_type=jax.ShapeDtypeStruct((B,), jnp.float32),
    mesh=plsc.VectorSubcoreMesh(core_axis_name="c", subcore_axis_name="s"),
    compiler_params=pltpu.CompilerParams(
        needs_layout_passes=False, use_tc_tiling_on_sc=False),
    scratch_types=[
        pltpu.VMEM((BPW,), jnp.int32),
        pltpu.VMEM((BPW,), jnp.int32),
        pltpu.VMEM((D, BPW), jnp.float32),
        pltpu.VMEM((D, BPW), jnp.float32),
        pltpu.VMEM((BPW,), jnp.float32),
        pltpu.SemaphoreType.DMA,
    ],
)
def _sc_gather_dot(users_r, items_r, ut_r, it_r, out_r,
                   uidx, iidx, ucols, icols, outv, sem):
    _gather_body(users_r, items_r, ut_r, it_r, out_r,
                 uidx, iidx, ucols, icols, outv, sem)


def _patch_input(table):
    return jnp.pad(table[NT * 128:, :].T, ((0, 0), (0, 64)))


def kernel(users, items, users_table, items_table):
    ubi, ibi = _sc_detile(users_table.T, items_table.T,
                          _patch_input(users_table),
                          _patch_input(items_table))
    users = users.astype(jnp.int32)
    items = items.astype(jnp.int32)
    ubase = (users >> 7) * 2048 + (users & 127)
    ibase = (items >> 7) * 2048 + (items & 127)
    out = _sc_gather_dot(ubase, ibase,
                         ubi.reshape(FLEN), ibi.reshape(FLEN))
    return out.reshape(B, 1)


# speedup vs baseline: 4.8630x; 4.8630x over previous
import functools

import jax
import jax.numpy as jnp
from jax import lax
from jax.experimental import pallas as pl
from jax.experimental.pallas import tpu as pltpu
from jax.experimental.pallas import tpu_sc as plsc

NC = 2
NS = 16
NW = NC * NS
L = 16

B = 16384
D = 16
BPW = B // NW
IDXC = 128
KCH = BPW // IDXC


def _body(users_r, items_r, ut_r, it_r, out_r,
          uidx, iidx, urows, irows, outv, sem):
    w = lax.axis_index("s") * NC + lax.axis_index("c")
    base = w * BPW

    pltpu.sync_copy(users_r.at[pl.ds(base, BPW)], uidx)
    pltpu.sync_copy(items_r.at[pl.ds(base, BPW)], iidx)

    handles = []
    for k in range(KCH):
        sl = pl.ds(k * IDXC, IDXC)
        handles.append(pltpu.async_copy(
            ut_r.at[uidx.at[sl]], urows.at[pl.ds(k * IDXC, IDXC), :], sem))
        handles.append(pltpu.async_copy(
            it_r.at[iidx.at[sl]], irows.at[pl.ds(k * IDXC, IDXC), :], sem))
    for h in handles:
        h.wait()

    iota = lax.iota(jnp.int32, L)

    def group(g, carry):
        row_idx = iota + g * L
        acc = jnp.zeros((L,), jnp.float32)
        for d in range(D):
            dcol = jnp.full((L,), d, jnp.int32)
            u = plsc.load_gather(urows, [row_idx, dcol])
            v = plsc.load_gather(irows, [row_idx, dcol])
            acc = acc + u * v
        outv[pl.ds(g * L, L)] = acc
        return carry

    lax.fori_loop(0, BPW // L, group, 0)

    pltpu.sync_copy(outv, out_r.at[pl.ds(base, BPW)])


@functools.partial(
    pl.kernel,
    out_type=jax.ShapeDtypeStruct((B,), jnp.float32),
    mesh=plsc.VectorSubcoreMesh(core_axis_name="c", subcore_axis_name="s"),
    compiler_params=pltpu.CompilerParams(
        needs_layout_passes=False, use_tc_tiling_on_sc=False),
    scratch_types=[
        pltpu.VMEM((BPW,), jnp.int32),
        pltpu.VMEM((BPW,), jnp.int32),
        pltpu.VMEM((BPW, D), jnp.float32),
        pltpu.VMEM((BPW, D), jnp.float32),
        pltpu.VMEM((BPW,), jnp.float32),
        pltpu.SemaphoreType.DMA,
    ],
)
def _sc_kernel(users_r, items_r, ut_r, it_r, out_r,
               uidx, iidx, urows, irows, outv, sem):
    _body(users_r, items_r, ut_r, it_r, out_r,
          uidx, iidx, urows, irows, outv, sem)


def kernel(users, items, users_table, items_table):
    out = _sc_kernel(users.astype(jnp.int32), items.astype(jnp.int32),
                     users_table, items_table)
    return out.reshape(B, 1)
